# Initial kernel scaffold; baseline (speedup 1.0000x reference)
import jax
import jax.numpy as jnp
from jax.experimental import pallas as pl


def kernel(h, edge_index, batch, params):
    raise NotImplementedError("write your pallas kernel here")


# speedup vs baseline: 3.3620x; 3.3620x over previous
import functools

import jax
import jax.numpy as jnp
from jax import lax
from jax.experimental import pallas as pl
from jax.experimental.pallas import tpu as pltpu
from jax.experimental.pallas import tpu_sc as plsc

_N = 10000
_E = 320000
_G = 16
_NP = 10240
_ZROW = 10048
_NZ = _NP - _ZROW
_NC = 5120
_NTILES = 16
_KB = 128
_NB = 160
_EP = _NTILES * _NB * _KB
_NBAT = _EP // _KB
_CPT = _NC // _NTILES
_BLK = 512
_NBLK = _NP // _BLK


@functools.lru_cache(maxsize=None)
def _make_sc_agg(edge_split):
  w = 128
  nb = _NB // 2 if edge_split else _NB
  mesh = plsc.VectorSubcoreMesh(core_axis_name="c", subcore_axis_name="s")

  @functools.partial(
      pl.kernel,
      out_type=jax.ShapeDtypeStruct((2, _NP, w), jnp.float32),
      mesh=mesh,
      scratch_types=[
          pltpu.VMEM((nb + 2, _KB), jnp.int32),
          pltpu.VMEM((nb, _KB), jnp.int32),
          pltpu.VMEM((_KB, w), jnp.float32),
          pltpu.VMEM((_KB, w), jnp.float32),
          pltpu.VMEM_SHARED((_NC, w), jnp.float32),
          pltpu.SemaphoreType.DMA,
          pltpu.SemaphoreType.DMA,
          pltpu.SemaphoreType.DMA,
          pltpu.SemaphoreType.DMA,
      ],
  )
  def agg(xflat, srcpp, dstpp, out, src_v, dst_v, buf0, buf1, acc, g0, g1,
          s0, s1):
    c = lax.axis_index("c")
    s = lax.axis_index("s")
    lanes = lax.iota(jnp.int32, 16)

    for p in range(2):
      if edge_split:
        base = (c * _NTILES + s) * nb
        pltpu.sync_copy(srcpp.at[p, pl.ds(base, nb)], src_v.at[pl.ds(0, nb)])
        pltpu.sync_copy(dstpp.at[p, pl.ds(base, nb)], dst_v)
      else:
        pltpu.sync_copy(srcpp.at[p, c, pl.ds(s * nb, nb)],
                        src_v.at[pl.ds(0, nb)])
        pltpu.sync_copy(dstpp.at[p, pl.ds(s * nb, nb)], dst_v)
      for r in (nb, nb + 1):
        for j in range(_KB // 16):
          src_v[r, pl.ds(j * 16, 16)] = lanes + (16 * j + _KB * (r - nb))

      zf = jnp.zeros((16,), jnp.float32)

      def _zero_row(r, _):
        for j in range(w // 16):
          buf0[r, pl.ds(j * 16, 16)] = zf
        return 0

      lax.fori_loop(0, _KB, _zero_row, 0)
      pltpu.sync_copy(buf0, acc.at[pl.ds(s * _CPT, _KB)])
      pltpu.sync_copy(buf0, acc.at[pl.ds(s * _CPT + _KB, _KB)])
      pltpu.sync_copy(buf0.at[pl.ds(0, _CPT - 2 * _KB)],
                      acc.at[pl.ds(s * _CPT + 2 * _KB, _CPT - 2 * _KB)])
      plsc.subcore_barrier()

      pltpu.async_copy(xflat.at[src_v.at[0]], buf0, g0)
      pltpu.async_copy(xflat.at[src_v.at[1]], buf1, g1)

      def _step(i, _):
        b0 = 2 * i
        pltpu.make_async_copy(xflat.at[src_v.at[b0]], buf0, g0).wait()
        pltpu.async_copy(buf0, acc.at[dst_v.at[b0]], s0, add=True)
        pltpu.make_async_copy(xflat.at[src_v.at[b0 + 1]], buf1, g1).wait()
        pltpu.async_copy(buf1, acc.at[dst_v.at[b0 + 1]], s1, add=True)
        pltpu.make_async_copy(buf0, acc.at[dst_v.at[b0]], s0).wait()
        pltpu.async_copy(xflat.at[src_v.at[b0 + 2]], buf0, g0)
        pltpu.make_async_copy(buf1, acc.at[dst_v.at[b0 + 1]], s1).wait()
        pltpu.async_copy(xflat.at[src_v.at[b0 + 3]], buf1, g1)
        return 0

      lax.fori_loop(0, nb // 2, _step, 0)
      pltpu.make_async_copy(xflat.at[src_v.at[0]], buf0, g0).wait()
      pltpu.make_async_copy(xflat.at[src_v.at[1]], buf1, g1).wait()
      plsc.subcore_barrier()

      pltpu.sync_copy(acc.at[pl.ds(s * _CPT, _CPT)],
                      out.at[c, pl.ds(p * _NC + s * _CPT, _CPT)])

  return agg


_P = jax.lax.Precision.HIGHEST


def _dot(a, b):
  return jnp.dot(a, b, precision=_P, preferred_element_type=jnp.float32)


def _bdot(a, b_bf16):
  return jnp.dot(a.astype(jnp.bfloat16), b_bf16,
                 preferred_element_type=jnp.float32)


def _zmask(j):
  grow = j * _BLK + lax.broadcasted_iota(jnp.int32, (_BLK, 1), 0)
  return grow >= _ZROW


def _mlp_bn(u, w1_ref, w2_ref, w3_ref, b_ref):
  t1 = jnp.maximum(
      (_bdot(u, w1_ref[...]) + b_ref[0:1, :]) * b_ref[1:2, :] + b_ref[2:3, :],
      0.0)
  t2 = jnp.maximum(
      (_bdot(t1, w2_ref[...]) + b_ref[3:4, :]) * b_ref[4:5, :] + b_ref[5:6, :],
      0.0)
  z = _bdot(t2, w3_ref[...]) + b_ref[6:7, :]
  return z * b_ref[7:8, :] + b_ref[8:9, :]


def _tc_layer1(h, agg, w1, w2, w3, bias):
  def body(h_ref, a_ref, w1_ref, w2_ref, w3_ref, b_ref, o_ref):
    u = h_ref[...] + a_ref[0] + a_ref[1]
    z = jnp.maximum(_mlp_bn(u, w1_ref, w2_ref, w3_ref, b_ref), 0.0)
    z = jnp.where(_zmask(pl.program_id(0)), 0.0, z)
    o_ref[0] = z[:, :128]
    o_ref[1] = z[:, 128:]

  return pl.pallas_call(
      body,
      grid=(_NBLK,),
      in_specs=[
          pl.BlockSpec((_BLK, 128), lambda j: (j, 0)),
          pl.BlockSpec((2, _BLK, 128), lambda j: (0, j, 0)),
          pl.BlockSpec((128, 256), lambda j: (0, 0)),
          pl.BlockSpec((256, 256), lambda j: (0, 0)),
          pl.BlockSpec((256, 256), lambda j: (0, 0)),
          pl.BlockSpec((9, 256), lambda j: (0, 0)),
      ],
      out_specs=pl.BlockSpec((2, _BLK, 128), lambda j: (0, j, 0)),
      out_shape=jax.ShapeDtypeStruct((2, _NP, 128), jnp.float32),
  )(h, agg, w1, w2, w3, bias)


def _tc_layer(x, agg, w1, w2, w3, bias):
  def body(x_ref, a_ref, w1_ref, w2_ref, w3_ref, b_ref, o_ref):
    xa, xb = x_ref[0], x_ref[1]
    u = jnp.concatenate([xa + a_ref[0], xb + a_ref[1]], axis=1)
    z = _mlp_bn(u, w1_ref, w2_ref, w3_ref, b_ref)
    zm = _zmask(pl.program_id(0))
    o_ref[0] = jnp.where(zm, 0.0, jnp.maximum(xa + z[:, :128], 0.0))
    o_ref[1] = jnp.where(zm, 0.0, jnp.maximum(xb + z[:, 128:], 0.0))

  return pl.pallas_call(
      body,
      grid=(_NBLK,),
      in_specs=[
          pl.BlockSpec((2, _BLK, 128), lambda j: (0, j, 0)),
          pl.BlockSpec((2, _BLK, 128), lambda j: (0, j, 0)),
          pl.BlockSpec((256, 256), lambda j: (0, 0)),
          pl.BlockSpec((256, 256), lambda j: (0, 0)),
          pl.BlockSpec((256, 256), lambda j: (0, 0)),
          pl.BlockSpec((9, 256), lambda j: (0, 0)),
      ],
      out_specs=pl.BlockSpec((2, _BLK, 128), lambda j: (0, j, 0)),
      out_shape=jax.ShapeDtypeStruct((2, _NP, 128), jnp.float32),
  )(x, agg, w1, w2, w3, bias)


def _tc_final(x, batch3, w1, w2, w3, bias):
  def body(x_ref, b_ref, w1_ref, w2_ref, w3_ref, bias_ref, o_ref, sums, cnts):
    j = pl.program_id(0)

    @pl.when(j == 0)
    def _():
      sums[...] = jnp.zeros_like(sums)
      cnts[...] = jnp.zeros_like(cnts)

    bb = b_ref[0, 0, :]
    oh = (bb[:, None] == lax.broadcasted_iota(jnp.int32, (1, _G), 1)
          ).astype(jnp.float32)
    xcat = jnp.concatenate([x_ref[0], x_ref[1]], axis=1)
    sums[...] += lax.dot_general(oh, xcat, (((0,), (0,)), ((), ())),
                                 precision=_P,
                                 preferred_element_type=jnp.float32)
    cnts[...] += lax.dot_general(oh, jnp.ones((_BLK, 128), jnp.float32),
                                 (((0,), (0,)), ((), ())),
                                 precision=_P,
                                 preferred_element_type=jnp.float32)

    @pl.when(j == _NBLK - 1)
    def _():
      cnt = jnp.maximum(cnts[...][:, 0:1], 1.0)
      p = sums[...] / cnt
      t1 = jnp.maximum(
          (_bdot(p, w1_ref[...]) + bias_ref[0:1, :]) * bias_ref[1:2, :]
          + bias_ref[2:3, :], 0.0)
      t2 = jnp.maximum(
          (_bdot(t1, w2_ref[...]) + bias_ref[3:4, :]) * bias_ref[4:5, :]
          + bias_ref[5:6, :], 0.0)
      o_ref[...] = _bdot(t2, w3_ref[...]) + bias_ref[6:7, 0:1]

  return pl.pallas_call(
      body,
      grid=(_NBLK,),
      in_specs=[
          pl.BlockSpec((2, _BLK, 128), lambda j: (0, j, 0)),
          pl.BlockSpec((1, 1, _BLK), lambda j: (j, 0, 0)),
          pl.BlockSpec((256, 128), lambda j: (0, 0)),
          pl.BlockSpec((128, 128), lambda j: (0, 0)),
          pl.BlockSpec((128, 1), lambda j: (0, 0)),
          pl.BlockSpec((7, 128), lambda j: (0, 0)),
      ],
      out_specs=pl.BlockSpec((_G, 1), lambda j: (0, 0)),
      out_shape=jax.ShapeDtypeStruct((_G, 1), jnp.float32),
      scratch_shapes=[
          pltpu.VMEM((_G, 256), jnp.float32),
          pltpu.VMEM((_G, 128), jnp.float32),
      ],
  )(x, batch3, w1, w2, w3, bias)


def _prep_mlp(p, gn=None, ben=None):
  rt = jnp.sqrt(jnp.float32(1.0 + 1e-5))
  dh = p["b1"].shape[0]
  rows = [p["b1"], p["g1"] / rt, p["be1"],
          p["b2"], p["g2"] / rt, p["be2"]]
  b3 = jnp.zeros((dh,), jnp.float32).at[: p["b3"].shape[0]].set(p["b3"])
  rows.append(b3)
  if gn is not None:
    rows.extend([gn / rt, ben])
  bias = jnp.stack(rows)
  bf = jnp.bfloat16
  return p["w1"].astype(bf), p["w2"].astype(bf), p["w3"].astype(bf), bias


def kernel(h, edge_index, batch, params):
  src, dst = edge_index[0], edge_index[1]
  pad = _EP - _E
  zfill = _ZROW + jnp.arange(_EP, dtype=jnp.int32) % _NZ
  srcp = jnp.concatenate([src, zfill[:pad]])
  dstp = jnp.concatenate([dst, jnp.zeros((pad,), jnp.int32)])
  sp, dp = [], []
  for p in range(2):
    inp = (dstp >= p * _NC) & (dstp < (p + 1) * _NC)
    sp.append(jnp.where(inp, srcp, zfill))
    dp.append(jnp.where(inp, dstp - p * _NC, 0))
  sp = jnp.stack(sp)
  dstpp = jnp.stack(dp).reshape(2, _NBAT, _KB)
  src1pp = sp.reshape(2, _NBAT, _KB)
  srcpp = jnp.stack([sp, sp + _NP], axis=1).reshape(2, 2, _NBAT, _KB)

  hp = jnp.pad(h, ((0, _NP - _N), (0, 0)))
  bp3 = jnp.pad(batch, (0, _NP - _N), constant_values=_G).reshape(
      _NBLK, 1, _BLK)

  agg1 = _make_sc_agg(True)(hp, src1pp, dstpp)
  x = _tc_layer1(hp, agg1,
                 *_prep_mlp(params["nn1"], params["norm1"]["g"],
                            params["norm1"]["be"]))
  sc128 = _make_sc_agg(False)
  for i in (2, 3, 4, 5):
    agg = sc128(x.reshape(2 * _NP, 128), srcpp, dstpp)
    x = _tc_layer(x, agg,
                  *_prep_mlp(params["nn%d" % i], params["norm%d" % i]["g"],
                             params["norm%d" % i]["be"]))
  return _tc_final(x, bp3, *_prep_mlp(params["final"]))


# speedup vs baseline: 3.6988x; 1.1002x over previous
import functools

import jax
import jax.numpy as jnp
from jax import lax
from jax.experimental import pallas as pl
from jax.experimental.pallas import tpu as pltpu
from jax.experimental.pallas import tpu_sc as plsc

_N = 10000
_E = 320000
_G = 16
_NP = 10240
_ZROW = 10048
_NZ = _NP - _ZROW
_NC = 5120
_CUT = 4992
_NTILES = 16
_KB = 128
_NB = 160
_EP = _NTILES * _NB * _KB
_NBAT = _EP // _KB
_CPT = _NC // _NTILES
_BLK = 512
_NBLK = _NP // _BLK


@functools.lru_cache(maxsize=None)
def _make_sc_agg(edge_split):
  w = 128
  nb = _NB // 2 if edge_split else _NB
  mesh = plsc.VectorSubcoreMesh(core_axis_name="c", subcore_axis_name="s")

  @functools.partial(
      pl.kernel,
      out_type=jax.ShapeDtypeStruct((2, _NP, w), jnp.float32),
      mesh=mesh,
      scratch_types=[
          pltpu.VMEM((nb + 2, _KB), jnp.int32),
          pltpu.VMEM((nb, _KB), jnp.int32),
          [pltpu.VMEM((_KB, w), jnp.float32) for _ in range(2)],
          pltpu.VMEM_SHARED((_NC, w), jnp.float32),
          [pltpu.SemaphoreType.DMA for _ in range(2)],
          [pltpu.SemaphoreType.DMA for _ in range(2)],
      ],
  )
  def agg(xflat, srcb, dstpp, out, src_v, dst_v, bufs, acc, gsems, ssems):
    c = lax.axis_index("c")
    s = lax.axis_index("s")
    lanes = lax.iota(jnp.int32, 16)

    if edge_split:
      ebase = (c * _NTILES + s) * nb
      pltpu.sync_copy(srcb.at[pl.ds(ebase, nb)], src_v.at[pl.ds(0, nb)])
    else:
      ebase = s * nb
      pltpu.sync_copy(srcb.at[c, pl.ds(ebase, nb)], src_v.at[pl.ds(0, nb)])
    for r in range(nb, nb + 2):
      for j in range(_KB // 16):
        src_v[r, pl.ds(j * 16, 16)] = lanes + (16 * j + _KB * (r - nb))

    for p in range(2):
      pltpu.sync_copy(dstpp.at[p, pl.ds(ebase, nb)], dst_v)

      zf = jnp.zeros((16,), jnp.float32)

      def _zero_row(r, _):
        for j in range(w // 16):
          bufs[0][r, pl.ds(j * 16, 16)] = zf
        return 0

      lax.fori_loop(0, _KB, _zero_row, 0)
      pltpu.sync_copy(bufs[0], acc.at[pl.ds(s * _CPT, _KB)])
      pltpu.sync_copy(bufs[0], acc.at[pl.ds(s * _CPT + _KB, _KB)])
      pltpu.sync_copy(bufs[0].at[pl.ds(0, _CPT - 2 * _KB)],
                      acc.at[pl.ds(s * _CPT + 2 * _KB, _CPT - 2 * _KB)])
      plsc.subcore_barrier()

      for k in range(2):
        pltpu.async_copy(xflat.at[src_v.at[k]], bufs[k], gsems[k])

      def _step(i, _):
        b0 = 2 * i
        for k in range(2):
          pltpu.make_async_copy(
              xflat.at[src_v.at[b0 + k]], bufs[k], gsems[k]).wait()
          pltpu.async_copy(bufs[k], acc.at[dst_v.at[b0 + k]], ssems[k],
                           add=True)
        for k in range(2):
          pltpu.make_async_copy(
              bufs[k], acc.at[dst_v.at[b0 + k]], ssems[k]).wait()
          pltpu.async_copy(xflat.at[src_v.at[b0 + 2 + k]], bufs[k], gsems[k])
        return 0

      lax.fori_loop(0, nb // 2, _step, 0)
      for k in range(2):
        pltpu.make_async_copy(xflat.at[src_v.at[k]], bufs[k], gsems[k]).wait()
      plsc.subcore_barrier()

      if p == 0:
        cpt0 = _CUT // _NTILES
        pltpu.sync_copy(acc.at[pl.ds(s * cpt0, cpt0)],
                        out.at[c, pl.ds(s * cpt0, cpt0)])
      else:
        pltpu.sync_copy(acc.at[pl.ds(s * _CPT, _CPT)],
                        out.at[c, pl.ds(_CUT + s * _CPT, _CPT)])

  return agg


_P = jax.lax.Precision.HIGHEST


def _dot(a, b):
  return jnp.dot(a, b, precision=_P, preferred_element_type=jnp.float32)


def _bdot(a, b_bf16):
  return jnp.dot(a.astype(jnp.bfloat16), b_bf16,
                 preferred_element_type=jnp.float32)


def _zmask(j):
  grow = j * _BLK + lax.broadcasted_iota(jnp.int32, (_BLK, 1), 0)
  return grow >= _ZROW


def _mlp_bn(u, w1_ref, w2_ref, w3_ref, b_ref):
  t1 = jnp.maximum(
      (_bdot(u, w1_ref[...]) + b_ref[0:1, :]) * b_ref[1:2, :] + b_ref[2:3, :],
      0.0)
  t2 = jnp.maximum(
      (_bdot(t1, w2_ref[...]) + b_ref[3:4, :]) * b_ref[4:5, :] + b_ref[5:6, :],
      0.0)
  z = _bdot(t2, w3_ref[...]) + b_ref[6:7, :]
  return z * b_ref[7:8, :] + b_ref[8:9, :]


def _tc_layer1(h, agg, w1, w2, w3, bias):
  def body(h_ref, a_ref, w1_ref, w2_ref, w3_ref, b_ref, o_ref):
    u = h_ref[...] + a_ref[0] + a_ref[1]
    z = jnp.maximum(_mlp_bn(u, w1_ref, w2_ref, w3_ref, b_ref), 0.0)
    z = jnp.where(_zmask(pl.program_id(0)), 0.0, z)
    o_ref[0] = z[:, :128]
    o_ref[1] = z[:, 128:]

  return pl.pallas_call(
      body,
      grid=(_NBLK,),
      in_specs=[
          pl.BlockSpec((_BLK, 128), lambda j: (j, 0)),
          pl.BlockSpec((2, _BLK, 128), lambda j: (0, j, 0)),
          pl.BlockSpec((128, 256), lambda j: (0, 0)),
          pl.BlockSpec((256, 256), lambda j: (0, 0)),
          pl.BlockSpec((256, 256), lambda j: (0, 0)),
          pl.BlockSpec((9, 256), lambda j: (0, 0)),
      ],
      out_specs=pl.BlockSpec((2, _BLK, 128), lambda j: (0, j, 0)),
      out_shape=jax.ShapeDtypeStruct((2, _NP, 128), jnp.float32),
  )(h, agg, w1, w2, w3, bias)


def _tc_layer(x, agg, w1, w2, w3, bias):
  def body(x_ref, a_ref, w1_ref, w2_ref, w3_ref, b_ref, o_ref):
    xa, xb = x_ref[0], x_ref[1]
    u = jnp.concatenate([xa + a_ref[0], xb + a_ref[1]], axis=1)
    z = _mlp_bn(u, w1_ref, w2_ref, w3_ref, b_ref)
    zm = _zmask(pl.program_id(0))
    o_ref[0] = jnp.where(zm, 0.0, jnp.maximum(xa + z[:, :128], 0.0))
    o_ref[1] = jnp.where(zm, 0.0, jnp.maximum(xb + z[:, 128:], 0.0))

  return pl.pallas_call(
      body,
      grid=(_NBLK,),
      in_specs=[
          pl.BlockSpec((2, _BLK, 128), lambda j: (0, j, 0)),
          pl.BlockSpec((2, _BLK, 128), lambda j: (0, j, 0)),
          pl.BlockSpec((256, 256), lambda j: (0, 0)),
          pl.BlockSpec((256, 256), lambda j: (0, 0)),
          pl.BlockSpec((256, 256), lambda j: (0, 0)),
          pl.BlockSpec((9, 256), lambda j: (0, 0)),
      ],
      out_specs=pl.BlockSpec((2, _BLK, 128), lambda j: (0, j, 0)),
      out_shape=jax.ShapeDtypeStruct((2, _NP, 128), jnp.float32),
  )(x, agg, w1, w2, w3, bias)


def _tc_final(x, batch3, w1, w2, w3, bias):
  def body(x_ref, b_ref, w1_ref, w2_ref, w3_ref, bias_ref, o_ref, sums, cnts):
    j = pl.program_id(0)

    @pl.when(j == 0)
    def _():
      sums[...] = jnp.zeros_like(sums)
      cnts[...] = jnp.zeros_like(cnts)

    bb = b_ref[0, 0, :]
    oh = (bb[:, None] == lax.broadcasted_iota(jnp.int32, (1, _G), 1)
          ).astype(jnp.float32)
    xcat = jnp.concatenate([x_ref[0], x_ref[1]], axis=1)
    sums[...] += lax.dot_general(oh, xcat, (((0,), (0,)), ((), ())),
                                 precision=_P,
                                 preferred_element_type=jnp.float32)
    cnts[...] += lax.dot_general(oh, jnp.ones((_BLK, 128), jnp.float32),
                                 (((0,), (0,)), ((), ())),
                                 precision=_P,
                                 preferred_element_type=jnp.float32)

    @pl.when(j == _NBLK - 1)
    def _():
      cnt = jnp.maximum(cnts[...][:, 0:1], 1.0)
      p = sums[...] / cnt
      t1 = jnp.maximum(
          (_bdot(p, w1_ref[...]) + bias_ref[0:1, :]) * bias_ref[1:2, :]
          + bias_ref[2:3, :], 0.0)
      t2 = jnp.maximum(
          (_bdot(t1, w2_ref[...]) + bias_ref[3:4, :]) * bias_ref[4:5, :]
          + bias_ref[5:6, :], 0.0)
      o_ref[...] = _bdot(t2, w3_ref[...]) + bias_ref[6:7, 0:1]

  return pl.pallas_call(
      body,
      grid=(_NBLK,),
      in_specs=[
          pl.BlockSpec((2, _BLK, 128), lambda j: (0, j, 0)),
          pl.BlockSpec((1, 1, _BLK), lambda j: (j, 0, 0)),
          pl.BlockSpec((256, 128), lambda j: (0, 0)),
          pl.BlockSpec((128, 128), lambda j: (0, 0)),
          pl.BlockSpec((128, 1), lambda j: (0, 0)),
          pl.BlockSpec((7, 128), lambda j: (0, 0)),
      ],
      out_specs=pl.BlockSpec((_G, 1), lambda j: (0, 0)),
      out_shape=jax.ShapeDtypeStruct((_G, 1), jnp.float32),
      scratch_shapes=[
          pltpu.VMEM((_G, 256), jnp.float32),
          pltpu.VMEM((_G, 128), jnp.float32),
      ],
  )(x, batch3, w1, w2, w3, bias)


def _prep_mlp(p, gn=None, ben=None):
  rt = jnp.sqrt(jnp.float32(1.0 + 1e-5))
  dh = p["b1"].shape[0]
  rows = [p["b1"], p["g1"] / rt, p["be1"],
          p["b2"], p["g2"] / rt, p["be2"]]
  b3 = jnp.zeros((dh,), jnp.float32).at[: p["b3"].shape[0]].set(p["b3"])
  rows.append(b3)
  if gn is not None:
    rows.extend([gn / rt, ben])
  bias = jnp.stack(rows)
  bf = jnp.bfloat16
  return p["w1"].astype(bf), p["w2"].astype(bf), p["w3"].astype(bf), bias


def kernel(h, edge_index, batch, params):
  src, dst = edge_index[0], edge_index[1]
  pad = _EP - _E
  fill = jnp.arange(_EP, dtype=jnp.int32)
  srcp = jnp.concatenate([src, _ZROW + fill[:pad] % _NZ])
  dstp = jnp.concatenate([dst, jnp.full((pad,), _NP, jnp.int32)])
  in0 = dstp < _CUT
  dp0 = jnp.where(in0, dstp, _CUT + fill % (_NC - _CUT))
  in1 = (dstp >= _CUT) & (dstp < _CUT + _NC)
  dp1 = jnp.where(in1, dstp - _CUT, (_N - _CUT) + 8 + fill % 104)
  dstpp = jnp.stack([dp0, dp1]).reshape(2, _NBAT, _KB)
  src1b = srcp.reshape(_NBAT, _KB)
  srcb = jnp.stack([srcp, srcp + _NP]).reshape(2, _NBAT, _KB)

  hp = jnp.pad(h, ((0, _NP - _N), (0, 0)))
  bp3 = jnp.pad(batch, (0, _NP - _N), constant_values=_G).reshape(
      _NBLK, 1, _BLK)

  agg1 = _make_sc_agg(True)(hp, src1b, dstpp)
  x = _tc_layer1(hp, agg1,
                 *_prep_mlp(params["nn1"], params["norm1"]["g"],
                            params["norm1"]["be"]))
  sc128 = _make_sc_agg(False)
  for i in (2, 3, 4, 5):
    agg = sc128(x.reshape(2 * _NP, 128), srcb, dstpp)
    x = _tc_layer(x, agg,
                  *_prep_mlp(params["nn%d" % i], params["norm%d" % i]["g"],
                             params["norm%d" % i]["be"]))
  return _tc_final(x, bp3, *_prep_mlp(params["final"]))
